# Initial kernel scaffold; baseline (speedup 1.0000x reference)
import jax
import jax.numpy as jnp
from jax.experimental import pallas as pl


def kernel(text, embeddings):
    raise NotImplementedError("write your pallas kernel here")


# speedup vs baseline: 13.8465x; 13.8465x over previous
import functools

import jax
import jax.numpy as jnp
from jax import lax
from jax.experimental import pallas as pl
from jax.experimental.pallas import tpu as pltpu
from jax.experimental.pallas import tpu_sc as plsc

D = 128
L = 200
NC = 2
NS = 16
NW = NC * NS
C0 = 128
C1 = L - C0
NLANE = 16
NACC = D // NLANE


def _sc_body(text_hbm, table_hbm, out_hbm,
             idxs_v, rows0_v, rows1_v, out_v, sem0, sem1):
    bpw = text_hbm.shape[0] // NW
    wid = lax.axis_index("s") * NC + lax.axis_index("c")
    base = wid * bpw

    pltpu.sync_copy(text_hbm.at[pl.ds(base, bpw)], idxs_v)

    def fire(i, rows_v, sem):
        pltpu.async_copy(table_hbm.at[idxs_v.at[i, pl.ds(0, C0)]],
                         rows_v.at[pl.ds(0, C0)], sem)
        pltpu.async_copy(table_hbm.at[idxs_v.at[i, pl.ds(C0, C1)]],
                         rows_v.at[pl.ds(C0, C1)], sem)

    def drain(rows_v, sem):
        pltpu.make_async_copy(table_hbm.at[pl.ds(0, L)], rows_v, sem).wait()

    scale = jnp.float32(1.0 / L)

    def reduce_into(i, rows_v):
        def body(t, accs):
            return tuple(accs[c] + rows_v[t, pl.ds(NLANE * c, NLANE)]
                         for c in range(NACC))
        accs = lax.fori_loop(
            0, L, body,
            tuple(jnp.zeros((NLANE,), jnp.float32) for _ in range(NACC)))
        for c in range(NACC):
            out_v[i, pl.ds(NLANE * c, NLANE)] = accs[c] * scale

    fire(0, rows0_v, sem0)

    def loop_body(j, carry):
        i0 = 2 * j
        fire(i0 + 1, rows1_v, sem1)
        drain(rows0_v, sem0)
        reduce_into(i0, rows0_v)

        @pl.when(i0 + 2 < bpw)
        def _():
            fire(i0 + 2, rows0_v, sem0)

        drain(rows1_v, sem1)
        reduce_into(i0 + 1, rows1_v)
        return carry

    lax.fori_loop(0, bpw // 2, loop_body, 0)
    pltpu.sync_copy(out_v, out_hbm.at[pl.ds(base, bpw)])


def kernel(text, embeddings):
    batch = text.shape[0]
    bpw = batch // NW
    run = functools.partial(
        pl.kernel,
        mesh=plsc.VectorSubcoreMesh(core_axis_name="c", subcore_axis_name="s"),
        out_type=jax.ShapeDtypeStruct((batch, D), jnp.float32),
        scratch_types=[
            pltpu.VMEM((bpw, L), jnp.int32),
            pltpu.VMEM((L, D), jnp.float32),
            pltpu.VMEM((L, D), jnp.float32),
            pltpu.VMEM((bpw, D), jnp.float32),
            pltpu.SemaphoreType.DMA,
            pltpu.SemaphoreType.DMA,
        ],
    )(_sc_body)
    return run(text.astype(jnp.int32), embeddings)


# speedup vs baseline: 17.1402x; 1.2379x over previous
import functools

import jax
import jax.numpy as jnp
from jax import lax
from jax.experimental import pallas as pl
from jax.experimental.pallas import tpu as pltpu
from jax.experimental.pallas import tpu_sc as plsc

D = 128
L = 200
NC = 2
NS = 16
NW = NC * NS
NLANE = 16
NACC = D // NLANE


def _sc_body(text_t_hbm, table_hbm, out_hbm, idxs_v, acc_v, sem):
    bpw = text_t_hbm.shape[1] // NW
    wid = lax.axis_index("s") * NC + lax.axis_index("c")
    base = wid * bpw

    pltpu.sync_copy(text_t_hbm.at[:, pl.ds(base, bpw)], idxs_v)

    zeros = jnp.zeros((NLANE,), jnp.float32)

    def zero_body(i, carry):
        for c in range(NACC):
            acc_v[i, pl.ds(NLANE * c, NLANE)] = zeros
        return carry

    lax.fori_loop(0, bpw, zero_body, 0)

    def fire_body(t, carry):
        pltpu.async_copy(table_hbm.at[idxs_v.at[t]], acc_v, sem, add=True)
        return carry

    lax.fori_loop(0, L, fire_body, 0)

    def drain_body(t, carry):
        pltpu.make_async_copy(table_hbm.at[pl.ds(0, bpw)], acc_v, sem).wait()
        return carry

    lax.fori_loop(0, L, drain_body, 0)

    scale = jnp.float32(1.0 / L)

    def scale_body(i, carry):
        for c in range(NACC):
            acc_v[i, pl.ds(NLANE * c, NLANE)] = (
                acc_v[i, pl.ds(NLANE * c, NLANE)] * scale)
        return carry

    lax.fori_loop(0, bpw, scale_body, 0)
    pltpu.sync_copy(acc_v, out_hbm.at[pl.ds(base, bpw)])


def kernel(text, embeddings):
    batch = text.shape[0]
    bpw = batch // NW
    run = functools.partial(
        pl.kernel,
        mesh=plsc.VectorSubcoreMesh(core_axis_name="c", subcore_axis_name="s"),
        out_type=jax.ShapeDtypeStruct((batch, D), jnp.float32),
        scratch_types=[
            pltpu.VMEM((L, bpw), jnp.int32),
            pltpu.VMEM((bpw, D), jnp.float32),
            pltpu.SemaphoreType.DMA,
        ],
    )(_sc_body)
    return run(text.astype(jnp.int32).T, embeddings)


# speedup vs baseline: 17.2103x; 1.0041x over previous
import functools

import jax
import jax.numpy as jnp
from jax import lax
from jax.experimental import pallas as pl
from jax.experimental.pallas import tpu as pltpu
from jax.experimental.pallas import tpu_sc as plsc

D = 128
L = 200
NC = 2
NS = 16
NW = NC * NS
NLANE = 16
NACC = D // NLANE


def _sc_body(text_t_hbm, table_hbm, out_hbm, idxs_v, acc_v, sem, isem):
    bpw = text_t_hbm.shape[1] // NW
    wid = lax.axis_index("s") * NC + lax.axis_index("c")
    base = wid * bpw

    idx_copy = pltpu.async_copy(text_t_hbm.at[:, pl.ds(base, bpw)], idxs_v,
                                isem)

    zeros = jnp.zeros((NLANE,), jnp.float32)

    def zero_body(i, carry):
        for c in range(NACC):
            acc_v[i, pl.ds(NLANE * c, NLANE)] = zeros
        return carry

    lax.fori_loop(0, bpw, zero_body, 0)
    idx_copy.wait()

    def fire_body(t, carry):
        pltpu.async_copy(table_hbm.at[idxs_v.at[t]], acc_v, sem, add=True)
        return carry

    lax.fori_loop(0, L, fire_body, 0)

    def drain_body(t, carry):
        pltpu.make_async_copy(table_hbm.at[pl.ds(0, bpw)], acc_v, sem).wait()
        return carry

    lax.fori_loop(0, L, drain_body, 0)

    scale = jnp.float32(1.0 / L)

    def scale_body(i, carry):
        for c in range(NACC):
            acc_v[i, pl.ds(NLANE * c, NLANE)] = (
                acc_v[i, pl.ds(NLANE * c, NLANE)] * scale)
        return carry

    lax.fori_loop(0, bpw, scale_body, 0)
    pltpu.sync_copy(acc_v, out_hbm.at[pl.ds(base, bpw)])


def kernel(text, embeddings):
    batch = text.shape[0]
    bpw = batch // NW
    run = functools.partial(
        pl.kernel,
        mesh=plsc.VectorSubcoreMesh(core_axis_name="c", subcore_axis_name="s"),
        out_type=jax.ShapeDtypeStruct((batch, D), jnp.float32),
        scratch_types=[
            pltpu.VMEM((L, bpw), jnp.int32),
            pltpu.VMEM((bpw, D), jnp.float32),
            pltpu.SemaphoreType.DMA,
            pltpu.SemaphoreType.DMA,
        ],
    )(_sc_body)
    return run(text.astype(jnp.int32).T, embeddings)
